# Initial kernel scaffold; baseline (speedup 1.0000x reference)
import jax
import jax.numpy as jnp
from jax.experimental import pallas as pl


def kernel(x, edge_index, edge_attr, batch, attn_w, attn_b, proj_w, proj_b):
    raise NotImplementedError("write your pallas kernel here")


# speedup vs baseline: 17.2980x; 17.2980x over previous
import functools

import jax
import jax.numpy as jnp
from jax.experimental import pallas as pl
from jax.experimental.pallas import tpu as pltpu

_N, _D, _G = 10000, 256, 128
_R = 1000
_NEG = -1e30


def _pool_body(x_ref, b_ref, aw_ref, ab_ref, pw_ref, pb_ref, out_ref,
               m_ref, d_ref, s_ref):
    i = pl.program_id(0)
    nsteps = pl.num_programs(0)

    @pl.when(i == 0)
    def _init():
        m_ref[...] = jnp.full((1, _G), _NEG, jnp.float32)
        d_ref[...] = jnp.zeros((1, _G), jnp.float32)
        s_ref[...] = jnp.zeros((_G, _D), jnp.float32)

    x = x_ref[...]
    a = jnp.dot(x, aw_ref[...],
                preferred_element_type=jnp.float32) + ab_ref[0, 0]
    b = b_ref[0, 0, :]
    gids = jax.lax.broadcasted_iota(jnp.int32, (_R, _G), 1)
    oh = (b[:, None] == gids)
    ohf = oh.astype(jnp.float32)

    tile_max = jnp.max(jnp.where(oh, a, _NEG), axis=0)
    m_old = m_ref[0, :]
    m_new = jnp.maximum(m_old, tile_max)
    alpha = jnp.exp(m_old - m_new)
    m_row = ohf @ m_new
    p = jnp.exp(a[:, 0] - m_row)

    d_ref[0, :] = d_ref[0, :] * alpha + jnp.sum(ohf * p[:, None], axis=0)
    contrib = jax.lax.dot_general(
        ohf * p[:, None], x,
        dimension_numbers=(((0,), (0,)), ((), ())),
        preferred_element_type=jnp.float32)
    s_ref[...] = s_ref[...] * alpha[:, None] + contrib

    m_ref[0, :] = m_new

    @pl.when(i == nsteps - 1)
    def _finish():
        pooled = s_ref[...] / (d_ref[0, :] + 1e-16)[:, None]
        out_ref[...] = jnp.dot(pooled, pw_ref[...],
                               preferred_element_type=jnp.float32) \
                       + pb_ref[0, :]


@functools.partial(jax.jit, static_argnames=("interpret",))
def _pool_tc(x, batch, attn_w, attn_b, proj_w, proj_b, interpret=False):
    nsteps = _N // _R
    batch3 = batch.reshape(nsteps, 1, _R)
    ab2 = attn_b.reshape(1, 1)
    pb2 = proj_b.reshape(1, _D)
    return pl.pallas_call(
        _pool_body,
        grid=(nsteps,),
        in_specs=[
            pl.BlockSpec((_R, _D), lambda i: (i, 0)),
            pl.BlockSpec((1, 1, _R), lambda i: (i, 0, 0)),
            pl.BlockSpec((_D, 1), lambda i: (0, 0)),
            pl.BlockSpec((1, 1), lambda i: (0, 0)),
            pl.BlockSpec((_D, _D), lambda i: (0, 0)),
            pl.BlockSpec((1, _D), lambda i: (0, 0)),
        ],
        out_specs=pl.BlockSpec((_G, _D), lambda i: (0, 0)),
        out_shape=jax.ShapeDtypeStruct((_G, _D), jnp.float32),
        scratch_shapes=[
            pltpu.VMEM((1, _G), jnp.float32),
            pltpu.VMEM((1, _G), jnp.float32),
            pltpu.VMEM((_G, _D), jnp.float32),
        ],
        compiler_params=pltpu.CompilerParams(
            dimension_semantics=("arbitrary",)),
        interpret=interpret,
    )(x, batch3, attn_w, ab2, proj_w, pb2)


def kernel(x, edge_index, edge_attr, batch, attn_w, attn_b, proj_w, proj_b):
    return _pool_tc(x, batch, attn_w, attn_b, proj_w, proj_b)
